# baseline (speedup 1.0000x reference)
import functools

import jax
import jax.numpy as jnp
from jax import lax
from jax.experimental import pallas as pl
from jax.experimental.pallas import tpu as pltpu
from jax.experimental.pallas import tpu_sc as plsc

L = 16
NW = 32
C = 8
N = 64
H = 256
W = 256
NHW = N * H * W
HO = 1024
WO = 1024
NPIX = HO * WO
PPT = NPIX // NW
P = 1024
NCHUNK = PPT // P
IR = P // 128


def _sc_body(table, q_hbm, u_hbm, v_hbm, out_hbm,
             q_v, u_v, v_v, wx_v, wy_v,
             i00, i01, i10, i11, b00, b01, b10, b11, ost, sem):
  wid = lax.axis_index("s") * 2 + lax.axis_index("c")
  tile_base = wid * PPT
  iot = lax.iota(jnp.int32, L)

  def chunk_body(ci, carry):
    g = tile_base + ci * P
    pltpu.sync_copy(q_hbm.at[pl.ds(g, P)], q_v)
    pltpu.sync_copy(u_hbm.at[pl.ds(g, P)], u_v)
    pltpu.sync_copy(v_hbm.at[pl.ds(g, P)], v_v)

    def idx_body(j, carry2):
      s = j * L
      u = u_v[pl.ds(s, L)] * float(W - 1)
      v = v_v[pl.ds(s, L)] * float(H - 1)
      x0 = jnp.minimum(u.astype(jnp.int32), W - 2)
      y0 = jnp.minimum(v.astype(jnp.int32), H - 2)
      wx_v[pl.ds(s, L)] = u - x0.astype(jnp.float32)
      wy_v[pl.ds(s, L)] = v - y0.astype(jnp.float32)
      base = q_v[pl.ds(s, L)] * (H * W) + y0 * W + x0
      r = j // 8
      cc = (j % 8) * L
      i00[r, pl.ds(cc, L)] = base
      i01[r, pl.ds(cc, L)] = base + 1
      i10[r, pl.ds(cc, L)] = base + W
      i11[r, pl.ds(cc, L)] = base + (W + 1)
      return carry2

    lax.fori_loop(0, P // L, idx_body, 0)

    cps = []
    for idx_ref, buf in ((i00, b00), (i01, b01), (i10, b10), (i11, b11)):
      for r in range(IR):
        cps.append(pltpu.async_copy(table.at[idx_ref.at[r]],
                                    buf.at[pl.ds(r * 128, 128), :], sem))
    for cp in cps:
      cp.wait()

    def comb_body(j, carry2):
      s = j * L
      row = s + iot
      wx = wx_v[pl.ds(s, L)]
      wy = wy_v[pl.ds(s, L)]
      for c in range(C):
        col = jnp.full((L,), c, jnp.int32)
        v00 = plsc.load_gather(b00, [row, col])
        v01 = plsc.load_gather(b01, [row, col])
        v10 = plsc.load_gather(b10, [row, col])
        v11 = plsc.load_gather(b11, [row, col])
        top = v00 + wx * (v01 - v00)
        bot = v10 + wx * (v11 - v10)
        ost[c, pl.ds(s, L)] = top + wy * (bot - top)
      return carry2

    lax.fori_loop(0, P // L, comb_body, 0)

    row = g // WO
    for c in range(C):
      pltpu.sync_copy(ost.at[c], out_hbm.at[0, c, row, :])
    return carry

  lax.fori_loop(0, NCHUNK, chunk_body, 0)


def _sc_resample(table, q, u, v):
  return pl.kernel(
      _sc_body,
      out_type=jax.ShapeDtypeStruct((1, C, HO, WO), jnp.float32),
      mesh=plsc.VectorSubcoreMesh(core_axis_name="c", subcore_axis_name="s"),
      compiler_params=pltpu.CompilerParams(
          needs_layout_passes=False, use_tc_tiling_on_sc=False),
      scratch_types=[
          pltpu.VMEM((P,), jnp.int32),
          pltpu.VMEM((P,), jnp.float32),
          pltpu.VMEM((P,), jnp.float32),
          pltpu.VMEM((P,), jnp.float32),
          pltpu.VMEM((P,), jnp.float32),
          pltpu.VMEM((IR, 128), jnp.int32),
          pltpu.VMEM((IR, 128), jnp.int32),
          pltpu.VMEM((IR, 128), jnp.int32),
          pltpu.VMEM((IR, 128), jnp.int32),
          pltpu.VMEM((P, C), jnp.float32),
          pltpu.VMEM((P, C), jnp.float32),
          pltpu.VMEM((P, C), jnp.float32),
          pltpu.VMEM((P, C), jnp.float32),
          pltpu.VMEM((C, P), jnp.float32),
          pltpu.SemaphoreType.DMA,
      ],
  )(table, q, u, v)


@jax.jit
def kernel(x, quad_idx, tex_uv):
  table = jnp.transpose(x[0], (1, 2, 3, 0)).reshape(NHW, C)
  q = quad_idx.reshape(-1).astype(jnp.int32)
  u = tex_uv[..., 0].reshape(-1)
  v = tex_uv[..., 1].reshape(-1)
  return _sc_resample(table, q, u, v)
